# baseline (speedup 1.0000x reference)
import functools

import jax
import jax.numpy as jnp
from jax import lax
from jax.experimental import pallas as pl
from jax.experimental.pallas import tpu as pltpu
from jax.experimental.pallas import tpu_sc as plsc

MAXLEN = 200
VOCAB = 100000
EMBED = 64
BATCH = 4096
LANES = 128

NC, NS = 2, 16
NW = NC * NS
B_PER_W = BATCH // NW

_mesh = plsc.VectorSubcoreMesh(core_axis_name="c", subcore_axis_name="s")


@functools.partial(
    pl.kernel,
    out_type=jax.ShapeDtypeStruct((MAXLEN, EMBED, BATCH), jnp.float32),
    mesh=_mesh,
    compiler_params=pltpu.CompilerParams(use_tc_tiling_on_sc=True, needs_layout_passes=False),
    scratch_types=[
        pltpu.VMEM((MAXLEN, B_PER_W), jnp.int32),
        pltpu.VMEM((MAXLEN, EMBED), jnp.float32),
        pltpu.VMEM((B_PER_W, LANES), jnp.float32),
        pltpu.VMEM((B_PER_W, LANES), jnp.float32),
        pltpu.VMEM((EMBED, B_PER_W), jnp.float32),
        pltpu.VMEM((EMBED, B_PER_W), jnp.float32),
        pltpu.SemaphoreType.DMA,
        pltpu.SemaphoreType.DMA,
        pltpu.SemaphoreType.DMA,
        pltpu.SemaphoreType.DMA,
    ],
)
def _sc_embed_t(idx_hbm, tok_hbm, pos_hbm, out_hbm,
                idx_all, pos_v, rows0, rows1, stg0, stg1,
                gsem0, gsem1, wsem0, wsem1):
    wid = lax.axis_index("s") * NC + lax.axis_index("c")
    b_base = wid * B_PER_W
    rows = (rows0, rows1)
    stg = (stg0, stg1)
    gsem = (gsem0, gsem1)
    wsem = (wsem0, wsem1)

    pltpu.sync_copy(idx_hbm.at[pl.ds(0, MAXLEN), pl.ds(b_base, B_PER_W)],
                    idx_all)
    pltpu.sync_copy(pos_hbm, pos_v)

    iota = jnp.arange(16, dtype=jnp.int32)

    def start_gather(l, p):
        pltpu.async_copy(tok_hbm.at[idx_all.at[l]], rows[p], gsem[p])

    def drain_gather(p):
        pltpu.make_async_copy(tok_hbm.at[pl.ds(0, B_PER_W)], rows[p],
                              gsem[p]).wait()

    def transpose_add_write(l, p):
        @plsc.parallel_loop(0, EMBED, unroll=4)
        def _t(d):
            dcol = jnp.full((16,), d, jnp.int32)
            posb = plsc.load_gather(pos_v, [jnp.full((16,), l, jnp.int32),
                                            dcol])
            for b0 in range(0, B_PER_W, 16):
                vals = plsc.load_gather(rows[p], [iota + b0, dcol])
                stg[p][d, pl.ds(b0, 16)] = vals + posb

        pltpu.async_copy(
            stg[p], out_hbm.at[l, pl.ds(0, EMBED), pl.ds(b_base, B_PER_W)],
            wsem[p])

    def drain_write(p):
        pltpu.make_async_copy(
            stg[p], out_hbm.at[0, pl.ds(0, EMBED), pl.ds(0, B_PER_W)],
            wsem[p]).wait()

    start_gather(0, 0)

    @pl.loop(0, MAXLEN // 2)
    def _pipeline(h):
        for b in (0, 1):
            l = h * 2 + b
            p, q = b, 1 - b

            @pl.when(l + 1 < MAXLEN)
            def _():
                start_gather(l + 1, q)
            drain_gather(p)

            @pl.when(l >= 2)
            def _():
                drain_write(p)
            transpose_add_write(l, p)

    drain_write(0)
    drain_write(1)


def kernel(x, token_table, pos_table):
    tok_padded = jnp.pad(token_table, ((0, 0), (0, LANES - EMBED)))
    out_t = _sc_embed_t(x.T.astype(jnp.int32), tok_padded, pos_table)
    return jnp.transpose(out_t, (2, 0, 1))


# speedup vs baseline: 1.6299x; 1.6299x over previous
import functools

import jax
import jax.numpy as jnp
from jax import lax
from jax.experimental import pallas as pl
from jax.experimental.pallas import tpu as pltpu
from jax.experimental.pallas import tpu_sc as plsc

MAXLEN = 200
VOCAB = 100000
EMBED = 64
BATCH = 4096
LPAIRS = MAXLEN // 2

NC, NS = 2, 16
NW = NC * NS
SEQ_PER_W = BATCH // NW
CHUNK0 = (0, 128)
CHUNK1 = (128, 72)
ADD_UNROLL = 4

_mesh = plsc.VectorSubcoreMesh(core_axis_name="c", subcore_axis_name="s")


@functools.partial(
    pl.kernel,
    out_type=jax.ShapeDtypeStruct((LPAIRS, BATCH, 128), jnp.float32),
    mesh=_mesh,
    compiler_params=pltpu.CompilerParams(use_tc_tiling_on_sc=False),
    scratch_types=[
        pltpu.VMEM((SEQ_PER_W, MAXLEN), jnp.int32),
        pltpu.VMEM((MAXLEN, EMBED), jnp.float32),
        pltpu.VMEM((CHUNK0[1], EMBED), jnp.float32),
        pltpu.VMEM((CHUNK1[1], EMBED), jnp.float32),
        pltpu.VMEM((CHUNK0[1] // 2, 128), jnp.float32),
        pltpu.VMEM((CHUNK1[1] // 2, 128), jnp.float32),
        pltpu.SemaphoreType.DMA,
        pltpu.SemaphoreType.DMA,
        pltpu.SemaphoreType.DMA,
        pltpu.SemaphoreType.DMA,
    ],
)
def _sc_embed(idx_hbm, tok_hbm, pos_hbm, out_hbm,
              idx_all, pos_v, rows0, rows1, stg0, stg1,
              gsem0, gsem1, wsem0, wsem1):
    wid = lax.axis_index("s") * NC + lax.axis_index("c")
    seq_base = wid * SEQ_PER_W
    rows = (rows0, rows1)
    stg = (stg0, stg1)
    gsem = (gsem0, gsem1)
    wsem = (wsem0, wsem1)
    chunk = (CHUNK0, CHUNK1)

    pltpu.sync_copy(idx_hbm.at[pl.ds(seq_base, SEQ_PER_W)], idx_all)
    pltpu.sync_copy(pos_hbm, pos_v)

    def start_gather(s, p):
        off, n = chunk[p]
        pltpu.async_copy(
            tok_hbm.at[idx_all.at[s, pl.ds(off, n)]], rows[p], gsem[p])

    def drain_gather(p):
        _, n = chunk[p]
        pltpu.make_async_copy(tok_hbm.at[pl.ds(0, n)], rows[p],
                              gsem[p]).wait()

    def add_and_write(s, p):
        off, n = chunk[p]

        @plsc.parallel_loop(0, n // 2, unroll=ADD_UNROLL)
        def _add(r2):
            for h in range(2):
                for c4 in range(EMBED // 16):
                    src = pl.ds(c4 * 16, 16)
                    dst = pl.ds(h * EMBED + c4 * 16, 16)
                    stg[p][r2, dst] = (rows[p][r2 * 2 + h, src]
                                       + pos_v[off + r2 * 2 + h, src])

        pltpu.async_copy(
            stg[p], out_hbm.at[pl.ds(off // 2, n // 2), seq_base + s],
            wsem[p])

    def drain_write(p):
        off, n = chunk[p]
        pltpu.make_async_copy(stg[p], out_hbm.at[pl.ds(off // 2, n // 2), 0],
                              wsem[p]).wait()

    start_gather(0, 0)

    @pl.loop(0, SEQ_PER_W)
    def _pipeline(h):
        start_gather(h, 1)
        drain_gather(0)

        @pl.when(h >= 1)
        def _():
            drain_write(0)
        add_and_write(h, 0)

        @pl.when(h + 1 < SEQ_PER_W)
        def _():
            start_gather(h + 1, 0)
        drain_gather(1)

        @pl.when(h >= 1)
        def _():
            drain_write(1)
        add_and_write(h, 1)

    drain_write(0)
    drain_write(1)


def _tc_relayout_body(in_ref, out_ref):
    t = in_ref[0]
    tt = jnp.transpose(t)
    out_ref[...] = tt.reshape(2, EMBED, BATCH)


def _tc_relayout(dense):
    return pl.pallas_call(
        _tc_relayout_body,
        grid=(LPAIRS,),
        in_specs=[pl.BlockSpec((1, BATCH, 128), lambda i: (i, 0, 0))],
        out_specs=pl.BlockSpec((2, EMBED, BATCH), lambda i: (i, 0, 0)),
        out_shape=jax.ShapeDtypeStruct((MAXLEN, EMBED, BATCH), jnp.float32),
    )(dense)


def kernel(x, token_table, pos_table):
    g = _sc_embed(x.astype(jnp.int32), token_table, pos_table)
    out_t = _tc_relayout(g)
    return out_t.transpose(2, 0, 1)
